# Initial kernel scaffold; baseline (speedup 1.0000x reference)
import jax
import jax.numpy as jnp
from jax.experimental import pallas as pl


def kernel(r, edge_index, phi, weights, amplitude, lengthscale):
    raise NotImplementedError("write your pallas kernel here")


# speedup vs baseline: 13.6679x; 13.6679x over previous
import functools

import jax
import jax.numpy as jnp
from jax import lax
from jax.experimental import pallas as pl
from jax.experimental.pallas import tpu as pltpu
from jax.experimental.pallas import tpu_sc as plsc

NBASIS = 128
EMB_NBASIS = 8
N_NODES = 10000
N_EDGES = 160000

NPAD = 10240
NROWS_N = NPAD // 128
EPAD = 163840
NROWS_E = EPAD // 128
EBLK = 256
EGRID = NROWS_E // EBLK
NREAL_ROWS = N_EDGES // 128

NW = 32
CHUNK = EPAD // NW
NVEC = CHUNK // 16
NACC_ROWS = NPAD // 16


def _softplus(z):
    return jnp.maximum(z, 0.0) + jnp.log1p(jnp.exp(-jnp.abs(z)))


def _sigmoid(z):
    t = jnp.exp(-jnp.abs(z))
    return jnp.where(z >= 0, 1.0 / (1.0 + t), t / (1.0 + t))


def _edge_kernel(rx, ry, rz, phi, amp, ls,
                 dens, dprime, pairp, ux, uy, uz, pairsum):
    i = pl.program_id(0)
    row = jax.lax.broadcasted_iota(jnp.int32, (EBLK, 128), 0) + i * EBLK
    mask = row < NREAL_ROWS

    x = jnp.sqrt(rx[...] ** 2 + ry[...] ** 2 + rz[...] ** 2)

    Lp = jnp.ones_like(x)
    S = _softplus(phi[0] * Lp)
    T = jnp.zeros_like(x)
    Lc = 1.0 - x
    Dc = -Lp

    def body(k, carry):
        Lp, Lc, Dc, S, T = carry
        kf = k.astype(jnp.float32)
        pk = phi[k]
        z = pk * Lc
        t = jnp.exp(-jnp.abs(z))
        sp = jnp.maximum(z, 0.0) + jnp.log1p(t)
        sg = jnp.where(z >= 0, 1.0 / (1.0 + t), t / (1.0 + t))
        S = S + sp
        T = T + pk * sg * Dc
        Ln = ((2.0 * kf + 1.0 - x) * Lc - kf * Lp) / (kf + 1.0)
        Dn = Dc - Lc
        return Lc, Ln, Dn, S, T

    Lp, Lc, Dc, S, T = lax.fori_loop(1, NBASIS, body, (Lp, Lc, Dc, S, T))

    emx = jnp.exp(-x)
    spa = _softplus(amp[0, 0])
    spl = _softplus(ls[0, 0])
    pair = spa * jnp.exp(-spl * x)
    invx = jnp.where(mask, 1.0 / x, 0.0)

    dens[...] = jnp.where(mask, emx * S, 0.0)
    dprime[...] = jnp.where(mask, emx * (T - S), 0.0)
    pairp[...] = jnp.where(mask, -spl * pair, 0.0)
    ux[...] = rx[...] * invx
    uy[...] = ry[...] * invx
    uz[...] = rz[...] * invx

    @pl.when(i == 0)
    def _():
        pairsum[0, 0] = 0.0
    pairsum[0, 0] += jnp.sum(jnp.where(mask, pair, 0.0))


def _run_edge_stage(rx, ry, rz, phi, amp, ls):
    eb = pl.BlockSpec((EBLK, 128), lambda i: (i, 0))
    sm = pl.BlockSpec(memory_space=pltpu.SMEM)
    out_shapes = ([jax.ShapeDtypeStruct((NROWS_E, 128), jnp.float32)] * 6
                  + [jax.ShapeDtypeStruct((1, 1), jnp.float32)])
    return pl.pallas_call(
        _edge_kernel,
        grid=(EGRID,),
        in_specs=[eb, eb, eb, sm, sm, sm],
        out_specs=[eb] * 6 + [pl.BlockSpec(memory_space=pltpu.SMEM)],
        out_shape=out_shapes,
    )(rx, ry, rz, phi, amp, ls)


def _sc_density_body(dens_hbm, dst_hbm, zeros_hbm, out_hbm,
                     dst_v, den_v, acc_v):
    cid = lax.axis_index("c")
    sid = lax.axis_index("s")
    wid = sid * 2 + cid
    base = wid * CHUNK

    pltpu.sync_copy(dst_hbm.at[pl.ds(base, CHUNK)], dst_v)
    pltpu.sync_copy(dens_hbm.at[pl.ds(base, CHUNK)], den_v)
    pltpu.sync_copy(zeros_hbm, acc_v)

    def body(i, _):
        s = pl.ds(i * 16, 16)
        plsc.addupdate_scatter(acc_v, [dst_v[s]], den_v[s])
        return 0

    lax.fori_loop(0, NVEC, body, 0)

    pltpu.sync_copy(acc_v, out_hbm.at[wid])


def _run_density_scatter(dens_flat, dst, zeros2d):
    mesh = plsc.VectorSubcoreMesh(core_axis_name="c", subcore_axis_name="s",
                                  num_cores=2, num_subcores=16)
    fn = functools.partial(
        pl.kernel,
        out_type=jax.ShapeDtypeStruct((NW, NPAD), jnp.float32),
        mesh=mesh,
        scratch_types=[
            pltpu.VMEM((CHUNK,), jnp.int32),
            pltpu.VMEM((CHUNK,), jnp.float32),
            pltpu.VMEM((NPAD,), jnp.float32),
        ],
        compiler_params=pltpu.CompilerParams(needs_layout_passes=False),
    )(_sc_density_body)
    return fn(dens_flat, dst, zeros2d)


def _node_kernel(rho32, w, psum, fp_out, energy):
    rho = rho32[0]
    for t in range(1, NW):
        rho = rho + rho32[t]

    Lp = jnp.ones_like(rho)
    curve = w[0] * Lp
    curvep = jnp.zeros_like(rho)
    Lc = 1.0 - rho
    Dc = -Lp
    wsum = w[0]
    for k in range(1, EMB_NBASIS):
        wk = w[k]
        wsum = wsum + wk
        curve = curve + wk * Lc
        curvep = curvep + wk * Dc
        kf = float(k)
        Ln = ((2.0 * kf + 1.0 - rho) * Lc - kf * Lp) / (kf + 1.0)
        Dn = Dc - Lc
        Lp, Lc, Dc = Lc, Ln, Dn

    F = _softplus(curve) - _softplus(wsum)
    fp_out[...] = _sigmoid(curve) * curvep
    energy[0, 0] = jnp.sum(F) + psum[0, 0]


def _run_node_stage(rho32, w, psum):
    sm = pl.BlockSpec(memory_space=pltpu.SMEM)
    return pl.pallas_call(
        _node_kernel,
        in_specs=[pl.BlockSpec((NW, NROWS_N, 128), lambda: (0, 0, 0)), sm, sm],
        out_specs=[pl.BlockSpec((NROWS_N, 128), lambda: (0, 0)), sm],
        out_shape=[jax.ShapeDtypeStruct((NROWS_N, 128), jnp.float32),
                   jax.ShapeDtypeStruct((1, 1), jnp.float32)],
    )(rho32, w, psum)


def _sc_force_body(fp_hbm, dp_hbm, pp_hbm, ux_hbm, uy_hbm, uz_hbm,
                   dst_hbm, src_hbm, zeros_hbm, outx_hbm, outy_hbm, outz_hbm,
                   fp_v, dp_v, pp_v, ux_v, uy_v, uz_v, dst_v, src_v,
                   accx, accy, accz):
    cid = lax.axis_index("c")
    sid = lax.axis_index("s")
    wid = sid * 2 + cid
    base = wid * CHUNK

    pltpu.sync_copy(fp_hbm, fp_v)
    pltpu.sync_copy(dp_hbm.at[pl.ds(base, CHUNK)], dp_v)
    pltpu.sync_copy(pp_hbm.at[pl.ds(base, CHUNK)], pp_v)
    pltpu.sync_copy(ux_hbm.at[pl.ds(base, CHUNK)], ux_v)
    pltpu.sync_copy(uy_hbm.at[pl.ds(base, CHUNK)], uy_v)
    pltpu.sync_copy(uz_hbm.at[pl.ds(base, CHUNK)], uz_v)
    pltpu.sync_copy(dst_hbm.at[pl.ds(base, CHUNK)], dst_v)
    pltpu.sync_copy(src_hbm.at[pl.ds(base, CHUNK)], src_v)
    pltpu.sync_copy(zeros_hbm, accx)
    pltpu.sync_copy(zeros_hbm, accy)
    pltpu.sync_copy(zeros_hbm, accz)

    def body(i, _):
        s = pl.ds(i * 16, 16)
        vd = dst_v[s]
        vs = src_v[s]
        fp = plsc.load_gather(fp_v, [vd])
        g = fp * dp_v[s] + pp_v[s]
        vx = g * ux_v[s]
        vy = g * uy_v[s]
        vz = g * uz_v[s]
        plsc.addupdate_scatter(accx, [vd], vx)
        plsc.addupdate_scatter(accx, [vs], -vx)
        plsc.addupdate_scatter(accy, [vd], vy)
        plsc.addupdate_scatter(accy, [vs], -vy)
        plsc.addupdate_scatter(accz, [vd], vz)
        plsc.addupdate_scatter(accz, [vs], -vz)
        return 0

    lax.fori_loop(0, NVEC, body, 0)

    pltpu.sync_copy(accx, outx_hbm.at[wid])
    pltpu.sync_copy(accy, outy_hbm.at[wid])
    pltpu.sync_copy(accz, outz_hbm.at[wid])


def _run_force_scatter(fp_flat, dp, pp, ux, uy, uz, dst, src, zeros2d):
    mesh = plsc.VectorSubcoreMesh(core_axis_name="c", subcore_axis_name="s",
                                  num_cores=2, num_subcores=16)
    vm_e_f = pltpu.VMEM((CHUNK,), jnp.float32)
    vm_e_i = pltpu.VMEM((CHUNK,), jnp.int32)
    vm_acc = pltpu.VMEM((NPAD,), jnp.float32)
    fn = functools.partial(
        pl.kernel,
        out_type=[jax.ShapeDtypeStruct((NW, NPAD), jnp.float32)] * 3,
        mesh=mesh,
        scratch_types=[
            pltpu.VMEM((NPAD,), jnp.float32),
            vm_e_f, vm_e_f, vm_e_f, vm_e_f, vm_e_f,
            vm_e_i, vm_e_i,
            vm_acc, vm_acc, vm_acc,
        ],
        compiler_params=pltpu.CompilerParams(needs_layout_passes=False),
    )(_sc_force_body)
    return fn(fp_flat, dp, pp, ux, uy, uz, dst, src, zeros2d)


def _combine_kernel(x, y, z, ox, oy, oz):
    ax, ay, az = x[0], y[0], z[0]
    for t in range(1, NW):
        ax = ax + x[t]
        ay = ay + y[t]
        az = az + z[t]
    ox[...] = ax
    oy[...] = ay
    oz[...] = az


def _run_force_combine(fx, fy, fz):
    ib = pl.BlockSpec((NW, NROWS_N, 128), lambda: (0, 0, 0))
    ob = pl.BlockSpec((NROWS_N, 128), lambda: (0, 0))
    return pl.pallas_call(
        _combine_kernel,
        in_specs=[ib, ib, ib],
        out_specs=[ob, ob, ob],
        out_shape=[jax.ShapeDtypeStruct((NROWS_N, 128), jnp.float32)] * 3,
    )(fx, fy, fz)


def kernel(r, edge_index, phi, weights, amplitude, lengthscale):
    epad = EPAD - N_EDGES
    rp = jnp.pad(r, ((0, epad), (0, 0)))
    rx = rp[:, 0].reshape(NROWS_E, 128)
    ry = rp[:, 1].reshape(NROWS_E, 128)
    rz = rp[:, 2].reshape(NROWS_E, 128)
    src = jnp.pad(edge_index[0], (0, epad))
    dst = jnp.pad(edge_index[1], (0, epad))
    amp = amplitude.reshape(1, 1)
    ls = lengthscale.reshape(1, 1)

    dens, dp, pp, ux, uy, uz, psum = _run_edge_stage(rx, ry, rz, phi, amp, ls)

    zeros1d = jnp.zeros((NPAD,), jnp.float32)

    rho32 = _run_density_scatter(dens.reshape(EPAD), dst, zeros1d)

    fp, energy = _run_node_stage(rho32.reshape(NW, NROWS_N, 128), weights,
                                 psum)

    fx32, fy32, fz32 = _run_force_scatter(
        fp.reshape(NPAD), dp.reshape(EPAD), pp.reshape(EPAD),
        ux.reshape(EPAD), uy.reshape(EPAD), uz.reshape(EPAD),
        dst, src, zeros1d)

    fx, fy, fz = _run_force_combine(fx32.reshape(NW, NROWS_N, 128),
                                    fy32.reshape(NW, NROWS_N, 128),
                                    fz32.reshape(NW, NROWS_N, 128))
    forces = jnp.stack([fx.reshape(NPAD), fy.reshape(NPAD),
                        fz.reshape(NPAD)], axis=1)[:N_NODES]
    return energy.reshape(()), forces
